# baseline (speedup 1.0000x reference)
import jax
import jax.numpy as jnp
from jax import lax
from jax.experimental import pallas as pl
from jax.experimental.pallas import tpu as pltpu
from jax.experimental.pallas import tpu_sc as plsc

N_USERS = 100000
N_ITEMS = 100000
D = 32
B = 4096
HIST = 20
HP = 32

NC = 2
NS = 16
NW = NC * NS
BPW = B // NW
UPG = 4
NHALF = 2
UPH = BPW // NHALF
NCH = UPH // UPG


def _worker_id():
    return lax.axis_index("s") * NC + lax.axis_index("c")


def _sc_gather_body(uids_hbm, iids_hbm, uw_hbm, ub_hbm, iw_hbm, ib_hbm,
                    eff_hbm, fbi_hbm, fbv_hbm,
                    uwp_out, iwg_out, ubg_out, ibg_out,
                    uid_v, iid_v, uw_v, iw_v,
                    ubrow_v, ibrow_v, ubrows_v, ibrows_v, ubg_v, ibg_v,
                    fbidx_v, fbval_v, fbflat_v, weff_v, uwp_v,
                    sem_a, sem_b, sem_c):
    wid = _worker_id()
    base = wid * BPW

    pltpu.sync_copy(uids_hbm.at[pl.ds(base, BPW)], uid_v)
    pltpu.sync_copy(iids_hbm.at[pl.ds(base, BPW)], iid_v)

    for g in range(BPW // 16):
        u16 = uid_v[pl.ds(g * 16, 16)]
        i16 = iid_v[pl.ds(g * 16, 16)]
        ubrow_v[pl.ds(g * 16, 16)] = jnp.right_shift(u16, 4)
        ibrow_v[pl.ds(g * 16, 16)] = jnp.right_shift(i16, 4)

    cp_fbi = pltpu.async_copy(fbi_hbm.at[uid_v], fbidx_v, sem_b)
    cp_uw = pltpu.async_copy(uw_hbm.at[uid_v], uw_v, sem_a)
    cp_fbv = pltpu.async_copy(fbv_hbm.at[uid_v], fbval_v, sem_a)
    cp_ub = pltpu.async_copy(ub_hbm.at[ubrow_v], ubrows_v, sem_a)
    cp_iw = pltpu.async_copy(iw_hbm.at[iid_v], iw_v, sem_a)
    cp_ib = pltpu.async_copy(ib_hbm.at[ibrow_v], ibrows_v, sem_a)

    cp_fbi.wait()
    cp_uw.wait()
    cp_fbv.wait()
    cp_ub.wait()
    cp_iw.wait()
    cp_ib.wait()

    lanes = lax.iota(jnp.int32, 16)
    for g in range(BPW // 16):
        u16 = uid_v[pl.ds(g * 16, 16)]
        i16 = iid_v[pl.ds(g * 16, 16)]
        uacc = jnp.zeros((16,), jnp.float32)
        iacc = jnp.zeros((16,), jnp.float32)
        for k in range(16):
            urow = ubrows_v[g * 16 + k, pl.ds(0, 16)]
            irow = ibrows_v[g * 16 + k, pl.ds(0, 16)]
            us = jnp.sum(jnp.where(lanes == jnp.bitwise_and(u16[k], 15),
                                   urow, 0.0))
            isx = jnp.sum(jnp.where(lanes == jnp.bitwise_and(i16[k], 15),
                                    irow, 0.0))
            uacc = jnp.where(lanes == k, us, uacc)
            iacc = jnp.where(lanes == k, isx, iacc)
        ubg_v[pl.ds(g * 16, 16)] = uacc
        ibg_v[pl.ds(g * 16, 16)] = iacc

    for half in range(NHALF):
        r0 = half * UPH

        def flat_body(r2, _):
            r = r0 + r2
            a = jnp.minimum(jnp.maximum(fbidx_v[r, pl.ds(0, 16)], 0),
                            N_ITEMS - 1)
            b = jnp.minimum(jnp.maximum(fbidx_v[r, pl.ds(16, 16)], 0),
                            N_ITEMS - 1)
            c = r2 // UPG
            o = (r2 % UPG) * HP
            fbflat_v[c, pl.ds(o, 16)] = a
            fbflat_v[c, pl.ds(o + 16, 16)] = b
            return _

        lax.fori_loop(0, UPH, flat_body, None)

        weff_cps = [
            pltpu.async_copy(
                eff_hbm.at[fbflat_v.at[c]],
                weff_v.at[pl.ds(c * UPG * HP, UPG * HP)], sem_c)
            for c in range(NCH)
        ]
        for cp in weff_cps:
            cp.wait()

        def row_body(r2, _):
            r = r0 + r2
            acc0 = uw_v[r, pl.ds(0, 16)]
            acc1 = uw_v[r, pl.ds(16, 16)]
            va = fbval_v[r, pl.ds(0, 16)]
            vb = fbval_v[r, pl.ds(16, 16)]
            p0 = r2 * HP
            for h in range(HP):
                val = va[h] if h < 16 else vb[h - 16]
                acc0 = acc0 + val * weff_v[p0 + h, pl.ds(0, 16)]
                acc1 = acc1 + val * weff_v[p0 + h, pl.ds(16, 16)]
            uwp_v[r, pl.ds(0, 16)] = acc0
            uwp_v[r, pl.ds(16, 16)] = acc1
            return _

        lax.fori_loop(0, UPH, row_body, None)

    pltpu.sync_copy(uwp_v, uwp_out.at[pl.ds(base, BPW)])
    pltpu.sync_copy(iw_v, iwg_out.at[pl.ds(base, BPW)])
    pltpu.sync_copy(ubg_v, ubg_out.at[pl.ds(base, BPW)])
    pltpu.sync_copy(ibg_v, ibg_out.at[pl.ds(base, BPW)])


def _sc_gather(user_ids, item_ids, user_weight, user_bias, item_weight,
               item_bias, item_rating_effect_weight, fb_indices, fb_values):
    mesh = plsc.VectorSubcoreMesh(core_axis_name="c", subcore_axis_name="s",
                                  num_cores=NC, num_subcores=NS)
    f = pl.kernel(
        _sc_gather_body,
        compiler_params=pltpu.CompilerParams(use_tc_tiling_on_sc=False,
                                             needs_layout_passes=False),
        out_type=(
            jax.ShapeDtypeStruct((B, D), jnp.float32),
            jax.ShapeDtypeStruct((B, D), jnp.float32),
            jax.ShapeDtypeStruct((B,), jnp.float32),
            jax.ShapeDtypeStruct((B,), jnp.float32),
        ),
        mesh=mesh,
        scratch_types=[
            pltpu.VMEM((BPW,), jnp.int32),
            pltpu.VMEM((BPW,), jnp.int32),
            pltpu.VMEM((BPW, D), jnp.float32),
            pltpu.VMEM((BPW, D), jnp.float32),
            pltpu.VMEM((BPW,), jnp.int32),
            pltpu.VMEM((BPW,), jnp.int32),
            pltpu.VMEM((BPW, 16), jnp.float32),
            pltpu.VMEM((BPW, 16), jnp.float32),
            pltpu.VMEM((BPW,), jnp.float32),
            pltpu.VMEM((BPW,), jnp.float32),
            pltpu.VMEM((BPW, HP), jnp.int32),
            pltpu.VMEM((BPW, HP), jnp.float32),
            pltpu.VMEM((NCH, UPG * HP), jnp.int32),
            pltpu.VMEM((UPH * HP, D), jnp.float32),
            pltpu.VMEM((BPW, D), jnp.float32),
            pltpu.SemaphoreType.DMA,
            pltpu.SemaphoreType.DMA,
            pltpu.SemaphoreType.DMA,
        ],
    )
    return f(user_ids, item_ids, user_weight, user_bias, item_weight,
             item_bias, item_rating_effect_weight, fb_indices, fb_values)


def _tc_rating_body(uwp_ref, iwg_ref, ubg_ref, ibgT_ref, bias_ref, out_ref):
    acc = lax.dot_general(uwp_ref[...], iwg_ref[...],
                          (((1,), (1,)), ((), ())),
                          preferred_element_type=jnp.float32)
    out_ref[...] = acc + ubg_ref[...] + ibgT_ref[...] + bias_ref[0]


def _tc_rating(uwp, iwg, ubg, ibgT, bias):
    bm = 1024
    bn = 1024
    grid = (B // bm, B // bn)
    return pl.pallas_call(
        _tc_rating_body,
        grid=grid,
        in_specs=[
            pl.BlockSpec((bm, D), lambda i, j: (i, 0)),
            pl.BlockSpec((bn, D), lambda i, j: (j, 0)),
            pl.BlockSpec((bm, 1), lambda i, j: (i, 0)),
            pl.BlockSpec((1, bn), lambda i, j: (0, j)),
            pl.BlockSpec(memory_space=pltpu.SMEM),
        ],
        out_specs=pl.BlockSpec((bm, bn), lambda i, j: (i, j)),
        out_shape=jax.ShapeDtypeStruct((B, B), jnp.float32),
    )(uwp, iwg, ubg, ibgT, bias)


@jax.jit
def kernel(user_ids, item_ids, user_weight, user_bias, item_weight, item_bias,
           bias, item_rating_effect_weight, fb_indices, fb_values):
    fbi32 = jnp.pad(fb_indices, ((0, 0), (0, HP - HIST)))
    fbv32 = jnp.pad(fb_values, ((0, 0), (0, HP - HIST)))
    uwp, iwg, ubg, ibg = _sc_gather(
        user_ids.astype(jnp.int32), item_ids.astype(jnp.int32),
        user_weight, user_bias.reshape(N_USERS // 16, 16),
        item_weight, item_bias.reshape(N_ITEMS // 16, 16),
        item_rating_effect_weight, fbi32, fbv32)
    return _tc_rating(uwp, iwg, ubg.reshape(B, 1), ibg.reshape(1, B), bias)
